# baseline (speedup 1.0000x reference)
import jax
import jax.numpy as jnp
from jax import lax
from jax.experimental import pallas as pl
from jax.experimental.pallas import tpu as pltpu
from jax.experimental.pallas import tpu_sc as plsc

B, F, V, K = 4096, 26, 100000, 64
NC, NS, L = 2, 16, 16
NW = NC * NS
RPW = B // NW
CH = 8
RPC = RPW // CH
IPC = RPC * F
SUB = 4
IPS = IPC // SUB


def _sc_body(idx_hbm, vals_hbm, lin_hbm, emb_hbm, trans_hbm, out_hbm,
             idx_v, vals_v, ebuf0, ebuf1, tbuf0, tbuf1, lbuf0, lbuf1,
             outb, sem0, sem1):
    ebufs, tbufs, lbufs = (ebuf0, ebuf1), (tbuf0, tbuf1), (lbuf0, lbuf1)
    wid = lax.axis_index("s") * NC + lax.axis_index("c")

    pltpu.sync_copy(idx_hbm.at[wid], idx_v)
    pltpu.sync_copy(vals_hbm.at[wid], vals_v)

    sems = (sem0, sem1)
    iota = lax.iota(jnp.int32, L)
    riota = iota * F
    zf = jnp.zeros((L,), jnp.float32)

    def issue(c, b):
        hs = []
        for j in range(SUB):
            s = SUB * c + j
            hs.append(pltpu.async_copy(
                emb_hbm.at[idx_v.at[s]], ebufs[b].at[pl.ds(j * IPS, IPS)],
                sems[b]))
            hs.append(pltpu.async_copy(
                trans_hbm.at[idx_v.at[s]], tbufs[b].at[pl.ds(j * IPS, IPS)],
                sems[b]))
            hs.append(pltpu.async_copy(
                lin_hbm.at[idx_v.at[s]], lbufs[b].at[pl.ds(j * IPS, IPS)],
                sems[b]))
        return hs

    def compute(c, b):
        eb = ebufs[b]
        tb = tbufs[b]
        lb = lbufs[b]
        vbase = riota + (c * IPC)

        def lin_body(f, carry):
            linb, ssum = carry
            v = plsc.load_gather(vals_v, [vbase + f])
            lg = plsc.load_gather(lb, [riota + f])
            return linb + v * lg, ssum + v
        linb, ssum = lax.fori_loop(0, F, lin_body, (zf, zf))

        def k_body(kk, carry):
            e2, et, t2, pe, pt, px = carry
            ck = jnp.full((L,), kk, jnp.int32)

            def f_body(f, c2):
                me, mt, pe, pt, px = c2
                r = riota + f
                v = plsc.load_gather(vals_v, [vbase + f])
                e = plsc.load_gather(eb, [r, ck])
                t = plsc.load_gather(tb, [r, ck])
                es = v * e
                ts = v * t
                return (me + es, mt + ts,
                        pe + es * e, pt + ts * t, px + es * t)

            me, mt, pe, pt, px = lax.fori_loop(
                0, F, f_body, (zf, zf, pe, pt, px))
            return (e2 + me * me, et + me * mt, t2 + mt * mt, pe, pt, px)

        e2, et, t2, pe, pt, px = lax.fori_loop(
            0, K, k_body, (zf, zf, zf, zf, zf, zf))

        preds = (linb
                 + 0.5 * ((2.0 * pe + pt + 2.0 * px) * ssum - 2.0 * e2
                          - 2.0 * et)
                 - 0.5 * t2)
        outb[pl.ds(c * RPC, RPC)] = preds

    pending = issue(0, 0)
    for c in range(CH):
        nxt = issue(c + 1, (c + 1) % 2) if c + 1 < CH else []
        for h in pending:
            h.wait()
        compute(c, c % 2)
        pending = nxt

    pltpu.sync_copy(outb, out_hbm.at[pl.ds(wid * RPW, RPW)])


def kernel(sparse_indices, sparse_values, var_linear, var_emb_factors,
           var_trans_factors):
    idx = sparse_indices.astype(jnp.int32).reshape(NW, CH * SUB, IPS)
    vals = sparse_values.astype(jnp.float32).reshape(NW, RPW * F)
    lin = var_linear.reshape(V)

    run = pl.kernel(
        _sc_body,
        out_type=jax.ShapeDtypeStruct((B,), jnp.float32),
        mesh=plsc.VectorSubcoreMesh(core_axis_name="c", subcore_axis_name="s"),
        compiler_params=pltpu.CompilerParams(
            needs_layout_passes=False, use_tc_tiling_on_sc=False),
        scratch_types=[
            pltpu.VMEM((CH * SUB, IPS), jnp.int32),
            pltpu.VMEM((RPW * F,), jnp.float32),
            pltpu.VMEM((IPC, K), jnp.float32),
            pltpu.VMEM((IPC, K), jnp.float32),
            pltpu.VMEM((IPC, K), jnp.float32),
            pltpu.VMEM((IPC, K), jnp.float32),
            pltpu.VMEM((IPC,), jnp.float32),
            pltpu.VMEM((IPC,), jnp.float32),
            pltpu.VMEM((RPW,), jnp.float32),
            pltpu.SemaphoreType.DMA,
            pltpu.SemaphoreType.DMA,
        ],
    )
    preds = run(idx, vals, lin, var_emb_factors, var_trans_factors)
    return preds.reshape(B, 1)


# speedup vs baseline: 2.2459x; 2.2459x over previous
import jax
import jax.numpy as jnp
from jax import lax
from jax.experimental import pallas as pl
from jax.experimental.pallas import tpu as pltpu
from jax.experimental.pallas import tpu_sc as plsc

B, F, V, K = 4096, 26, 100000, 64
NC, NS, L = 2, 16, 16
NW = NC * NS
RPW = B // NW
CH = 8
RPC = RPW // CH
IPC = RPC * F
SUB = 4
IPS = IPC // SUB
VROW = 32


def _sc_body(idx_hbm, vals_hbm, lin_hbm, emb_hbm, trans_hbm, out_hbm,
             idx_v, vals_v, ebuf0, ebuf1, tbuf0, tbuf1, lbuf0, lbuf1,
             outb, sem0, sem1):
    ebufs, tbufs, lbufs = (ebuf0, ebuf1), (tbuf0, tbuf1), (lbuf0, lbuf1)
    wid = lax.axis_index("s") * NC + lax.axis_index("c")

    pltpu.sync_copy(idx_hbm.at[wid], idx_v)
    pltpu.sync_copy(vals_hbm.at[wid], vals_v)

    sems = (sem0, sem1)
    zf = jnp.zeros((L,), jnp.float32)

    def issue(c, b):
        hs = []
        for j in range(SUB):
            s = SUB * c + j
            hs.append(pltpu.async_copy(
                emb_hbm.at[idx_v.at[s]], ebufs[b].at[pl.ds(j * IPS, IPS)],
                sems[b]))
            hs.append(pltpu.async_copy(
                trans_hbm.at[idx_v.at[s]], tbufs[b].at[pl.ds(j * IPS, IPS)],
                sems[b]))
            hs.append(pltpu.async_copy(
                lin_hbm.at[idx_v.at[s]], lbufs[b].at[pl.ds(j * IPS, IPS)],
                sems[b]))
        return hs

    iota = lax.iota(jnp.int32, L)
    riota26 = iota * F
    riota32 = iota * VROW

    def compute(c, b):
        eb = ebufs[b]
        tb = tbufs[b]
        lb = lbufs[b]

        ssum_vec = zf
        lin_vec = zf
        vgbase = riota32 + (c * RPC * VROW)
        for f in range(F):
            v = plsc.load_gather(vals_v, [vgbase + f])
            lg = plsc.load_gather(lb, [riota26 + f])
            ssum_vec = ssum_vec + v
            lin_vec = lin_vec + v * lg

        def row_body(rr, pred_vec):
            vbase = (c * RPC + rr) * VROW
            vrow0 = vals_v[pl.ds(vbase, L)]
            vrow1 = vals_v[pl.ds(vbase + L, L)]
            base = rr * F

            me = [zf, zf, zf, zf]
            mu = [zf, zf, zf, zf]
            pev = zf
            puv = zf
            for f in range(F):
                vrow = vrow0 if f < L else vrow1
                v = vrow.at[jnp.full((L,), f % L, jnp.int32)].get(
                    mode="promise_in_bounds")
                j = base + f
                for kc in range(4):
                    e = eb[j, pl.ds(kc * L, L)]
                    t = tb[j, pl.ds(kc * L, L)]
                    u = e + t
                    es = v * e
                    us = v * u
                    me[kc] = me[kc] + es
                    mu[kc] = mu[kc] + us
                    pev = pev + es * e
                    puv = puv + us * u

            ridx = jnp.full((L,), rr, jnp.int32)
            srow = ssum_vec.at[ridx].get(mode="promise_in_bounds")
            linr = lin_vec.at[ridx].get(mode="promise_in_bounds")
            vec = ((pev + puv) * srow
                   - me[0] * me[0] - me[1] * me[1]
                   - me[2] * me[2] - me[3] * me[3]
                   - mu[0] * mu[0] - mu[1] * mu[1]
                   - mu[2] * mu[2] - mu[3] * mu[3])
            red = lax.reduce_sum(vec, axes=(0,))
            return jnp.where(iota == rr, linr + 0.5 * red, pred_vec)

        preds = lax.fori_loop(0, RPC, row_body, zf)
        outb[pl.ds(c * RPC, RPC)] = preds

    pending = issue(0, 0)
    for c in range(CH):
        nxt = issue(c + 1, (c + 1) % 2) if c + 1 < CH else []
        for h in pending:
            h.wait()
        compute(c, c % 2)
        pending = nxt

    pltpu.sync_copy(outb, out_hbm.at[pl.ds(wid * RPW, RPW)])


def kernel(sparse_indices, sparse_values, var_linear, var_emb_factors,
           var_trans_factors):
    idx = sparse_indices.astype(jnp.int32).reshape(NW, CH * SUB, IPS)
    vals = jnp.pad(sparse_values.astype(jnp.float32),
                   ((0, 0), (0, VROW - F))).reshape(NW, RPW * VROW)
    lin = var_linear.reshape(V)

    run = pl.kernel(
        _sc_body,
        out_type=jax.ShapeDtypeStruct((B,), jnp.float32),
        mesh=plsc.VectorSubcoreMesh(core_axis_name="c", subcore_axis_name="s"),
        compiler_params=pltpu.CompilerParams(
            needs_layout_passes=False, use_tc_tiling_on_sc=False),
        scratch_types=[
            pltpu.VMEM((CH * SUB, IPS), jnp.int32),
            pltpu.VMEM((RPW * VROW,), jnp.float32),
            pltpu.VMEM((IPC, K), jnp.float32),
            pltpu.VMEM((IPC, K), jnp.float32),
            pltpu.VMEM((IPC, K), jnp.float32),
            pltpu.VMEM((IPC, K), jnp.float32),
            pltpu.VMEM((IPC,), jnp.float32),
            pltpu.VMEM((IPC,), jnp.float32),
            pltpu.VMEM((RPW,), jnp.float32),
            pltpu.SemaphoreType.DMA,
            pltpu.SemaphoreType.DMA,
        ],
    )
    preds = run(idx, vals, lin, var_emb_factors, var_trans_factors)
    return preds.reshape(B, 1)


# speedup vs baseline: 2.2515x; 1.0025x over previous
import jax
import jax.numpy as jnp
from jax import lax
from jax.experimental import pallas as pl
from jax.experimental.pallas import tpu as pltpu
from jax.experimental.pallas import tpu_sc as plsc

B, F, V, K = 4096, 26, 100000, 64
NC, NS, L = 2, 16, 16
NW = NC * NS
RPW = B // NW
CH = 8
RPC = RPW // CH
IPC = RPC * F
SUB = 4
IPS = IPC // SUB
VROW = 32


def _sc_body(idx_hbm, vals_hbm, lin_hbm, emb_hbm, trans_hbm, out_hbm,
             idx_v, vals_v, ebuf0, ebuf1, tbuf0, tbuf1, lbuf0, lbuf1,
             outb, sem0, sem1):
    ebufs, tbufs, lbufs = (ebuf0, ebuf1), (tbuf0, tbuf1), (lbuf0, lbuf1)
    wid = lax.axis_index("s") * NC + lax.axis_index("c")

    pltpu.sync_copy(idx_hbm.at[wid], idx_v)
    pltpu.sync_copy(vals_hbm.at[wid], vals_v)

    sems = (sem0, sem1)
    zf = jnp.zeros((L,), jnp.float32)

    def issue(c, b):
        hs = []
        for j in range(SUB):
            s = SUB * c + j
            hs.append(pltpu.async_copy(
                emb_hbm.at[idx_v.at[s]], ebufs[b].at[pl.ds(j * IPS, IPS)],
                sems[b]))
            hs.append(pltpu.async_copy(
                trans_hbm.at[idx_v.at[s]], tbufs[b].at[pl.ds(j * IPS, IPS)],
                sems[b]))
            hs.append(pltpu.async_copy(
                lin_hbm.at[idx_v.at[s]], lbufs[b].at[pl.ds(j * IPS, IPS)],
                sems[b]))
        return hs

    iota = lax.iota(jnp.int32, L)
    riota26 = iota * F
    riota32 = iota * VROW

    def compute(c, b):
        eb = ebufs[b]
        tb = tbufs[b]
        lb = lbufs[b]

        ssum_vec = zf
        lin_vec = zf
        vgbase = riota32 + (c * RPC * VROW)
        for f in range(F):
            v = plsc.load_gather(vals_v, [vgbase + f])
            lg = plsc.load_gather(lb, [riota26 + f])
            ssum_vec = ssum_vec + v
            lin_vec = lin_vec + v * lg

        zi = jnp.zeros((L,), jnp.int32)
        rot = jnp.bitwise_and(iota + 1, L - 1)

        def row_body(rr, pred_vec):
            vbase = (c * RPC + rr) * VROW
            vrow0 = vals_v[pl.ds(vbase, L)]
            vrow1 = vals_v[pl.ds(vbase + L, L)]
            base = rr * F

            me = [None] * 4
            mu = [None] * 4
            pev = None
            puv = None

            def acc(a, x):
                return x if a is None else a + x

            vrow = vrow0
            for f in range(F):
                if f == L:
                    vrow = vrow1
                v = vrow.at[zi].get(mode="promise_in_bounds")
                vrow = vrow.at[rot].get(mode="promise_in_bounds")
                j = base + f
                for kc in range(4):
                    e = eb[j, pl.ds(kc * L, L)]
                    t = tb[j, pl.ds(kc * L, L)]
                    u = e + t
                    es = v * e
                    us = v * u
                    me[kc] = acc(me[kc], es)
                    mu[kc] = acc(mu[kc], us)
                    pev = acc(pev, es * e)
                    puv = acc(puv, us * u)

            ridx = jnp.full((L,), rr, jnp.int32)
            srow = ssum_vec.at[ridx].get(mode="promise_in_bounds")
            linr = lin_vec.at[ridx].get(mode="promise_in_bounds")
            vec = ((pev + puv) * srow
                   - me[0] * me[0] - me[1] * me[1]
                   - me[2] * me[2] - me[3] * me[3]
                   - mu[0] * mu[0] - mu[1] * mu[1]
                   - mu[2] * mu[2] - mu[3] * mu[3])
            red = lax.reduce_sum(vec, axes=(0,))
            return jnp.where(iota == rr, linr + 0.5 * red, pred_vec)

        preds = lax.fori_loop(0, RPC, row_body, zf)
        outb[pl.ds(c * RPC, RPC)] = preds

    pending = issue(0, 0)
    for c in range(CH):
        nxt = issue(c + 1, (c + 1) % 2) if c + 1 < CH else []
        for h in pending:
            h.wait()
        compute(c, c % 2)
        pending = nxt

    pltpu.sync_copy(outb, out_hbm.at[pl.ds(wid * RPW, RPW)])


def kernel(sparse_indices, sparse_values, var_linear, var_emb_factors,
           var_trans_factors):
    idx = sparse_indices.astype(jnp.int32).reshape(NW, CH * SUB, IPS)
    vals = jnp.pad(sparse_values.astype(jnp.float32),
                   ((0, 0), (0, VROW - F))).reshape(NW, RPW * VROW)
    lin = var_linear.reshape(V)

    run = pl.kernel(
        _sc_body,
        out_type=jax.ShapeDtypeStruct((B,), jnp.float32),
        mesh=plsc.VectorSubcoreMesh(core_axis_name="c", subcore_axis_name="s"),
        compiler_params=pltpu.CompilerParams(
            needs_layout_passes=False, use_tc_tiling_on_sc=False),
        scratch_types=[
            pltpu.VMEM((CH * SUB, IPS), jnp.int32),
            pltpu.VMEM((RPW * VROW,), jnp.float32),
            pltpu.VMEM((IPC, K), jnp.float32),
            pltpu.VMEM((IPC, K), jnp.float32),
            pltpu.VMEM((IPC, K), jnp.float32),
            pltpu.VMEM((IPC, K), jnp.float32),
            pltpu.VMEM((IPC,), jnp.float32),
            pltpu.VMEM((IPC,), jnp.float32),
            pltpu.VMEM((RPW,), jnp.float32),
            pltpu.SemaphoreType.DMA,
            pltpu.SemaphoreType.DMA,
        ],
    )
    preds = run(idx, vals, lin, var_emb_factors, var_trans_factors)
    return preds.reshape(B, 1)


# speedup vs baseline: 2.5378x; 1.1271x over previous
import jax
import jax.numpy as jnp
from jax import lax
from jax.experimental import pallas as pl
from jax.experimental.pallas import tpu as pltpu
from jax.experimental.pallas import tpu_sc as plsc

B, F, V, K = 4096, 26, 100000, 64
NC, NS, L = 2, 16, 16
NW = NC * NS
RPW = B // NW
CH = 8
RPC = RPW // CH
IPC = RPC * F
SUB = 4
IPS = IPC // SUB
VROW = 32
CW = 2 * K


def _sc_body(idx_hbm, vals_hbm, lin_hbm, cat_hbm, out_hbm,
             idx_v, vals_v, cbuf0, cbuf1, lbuf0, lbuf1,
             outb, sem0, sem1):
    cbufs, lbufs, sems = (cbuf0, cbuf1), (lbuf0, lbuf1), (sem0, sem1)
    wid = lax.axis_index("s") * NC + lax.axis_index("c")

    pltpu.sync_copy(idx_hbm.at[wid], idx_v)
    pltpu.sync_copy(vals_hbm.at[wid], vals_v)

    zf = jnp.zeros((L,), jnp.float32)
    iota = lax.iota(jnp.int32, L)
    riota26 = iota * F
    riota32 = iota * VROW

    def issue(c, b):
        hs = []
        for j in range(SUB):
            isl = idx_v.at[pl.ds((SUB * c + j) * IPS, IPS)]
            hs.append(pltpu.async_copy(
                cat_hbm.at[isl], cbufs[b].at[pl.ds(j * IPS, IPS)], sems[b]))
            hs.append(pltpu.async_copy(
                lin_hbm.at[isl], lbufs[b].at[pl.ds(j * IPS, IPS)], sems[b]))
        return hs

    def compute(c, b):
        cb = cbufs[b]
        lb = lbufs[b]

        ssum_vec = zf
        lin_vec = zf
        vgbase = riota32 + (c * RPC * VROW)
        for f in range(F):
            v = plsc.load_gather(vals_v, [vgbase + f])
            lg = plsc.load_gather(lb, [riota26 + f])
            ssum_vec = ssum_vec + v
            lin_vec = lin_vec + v * lg

        zi = jnp.zeros((L,), jnp.int32)
        rot = jnp.bitwise_and(iota + 1, L - 1)

        def row_body(rr, pred_vec):
            vbase = (c * RPC + rr) * VROW
            vrow0 = vals_v[pl.ds(vbase, L)]
            vrow1 = vals_v[pl.ds(vbase + L, L)]
            base = rr * F

            me = [None] * 4
            mu = [None] * 4
            pev = None
            puv = None

            def acc(a, x):
                return x if a is None else a + x

            vrow = vrow0
            for f in range(F):
                if f == L:
                    vrow = vrow1
                v = vrow.at[zi].get(mode="promise_in_bounds")
                vrow = vrow.at[rot].get(mode="promise_in_bounds")
                j = base + f
                for kc in range(4):
                    e = cb[j, pl.ds(kc * L, L)]
                    t = cb[j, pl.ds(K + kc * L, L)]
                    u = e + t
                    es = v * e
                    us = v * u
                    me[kc] = acc(me[kc], es)
                    mu[kc] = acc(mu[kc], us)
                    pev = acc(pev, es * e)
                    puv = acc(puv, us * u)

            ridx = jnp.full((L,), rr, jnp.int32)
            srow = ssum_vec.at[ridx].get(mode="promise_in_bounds")
            linr = lin_vec.at[ridx].get(mode="promise_in_bounds")
            vec = ((pev + puv) * srow
                   - me[0] * me[0] - me[1] * me[1]
                   - me[2] * me[2] - me[3] * me[3]
                   - mu[0] * mu[0] - mu[1] * mu[1]
                   - mu[2] * mu[2] - mu[3] * mu[3])
            red = lax.reduce_sum(vec, axes=(0,))
            return jnp.where(iota == rr, linr + 0.5 * red, pred_vec)

        preds = lax.fori_loop(0, RPC, row_body, zf)
        outb[pl.ds(c * RPC, RPC)] = preds

    pending = issue(0, 0)
    for c in range(CH):
        nxt = issue(c + 1, (c + 1) % 2) if c + 1 < CH else []
        for h in pending:
            h.wait()
        compute(c, c % 2)
        pending = nxt

    pltpu.sync_copy(outb, out_hbm.at[pl.ds(wid * RPW, RPW)])


def kernel(sparse_indices, sparse_values, var_linear, var_emb_factors,
           var_trans_factors):
    idx = sparse_indices.astype(jnp.int32).reshape(NW, RPW * F)
    vals = jnp.pad(sparse_values.astype(jnp.float32),
                   ((0, 0), (0, VROW - F))).reshape(NW, RPW * VROW)
    lin = var_linear.reshape(V)
    cat = jnp.concatenate([var_emb_factors, var_trans_factors], axis=1)

    run = pl.kernel(
        _sc_body,
        out_type=jax.ShapeDtypeStruct((B,), jnp.float32),
        mesh=plsc.VectorSubcoreMesh(core_axis_name="c", subcore_axis_name="s"),
        compiler_params=pltpu.CompilerParams(
            needs_layout_passes=False, use_tc_tiling_on_sc=False),
        scratch_types=[
            pltpu.VMEM((RPW * F,), jnp.int32),
            pltpu.VMEM((RPW * VROW,), jnp.float32),
            pltpu.VMEM((IPC, CW), jnp.float32),
            pltpu.VMEM((IPC, CW), jnp.float32),
            pltpu.VMEM((IPC,), jnp.float32),
            pltpu.VMEM((IPC,), jnp.float32),
            pltpu.VMEM((RPW,), jnp.float32),
            pltpu.SemaphoreType.DMA,
            pltpu.SemaphoreType.DMA,
        ],
    )
    preds = run(idx, vals, lin, cat)
    return preds.reshape(B, 1)


# speedup vs baseline: 3.2182x; 1.2681x over previous
import jax
import jax.numpy as jnp
from jax import lax
from jax.experimental import pallas as pl
from jax.experimental.pallas import tpu as pltpu
from jax.experimental.pallas import tpu_sc as plsc

B, F, V, K = 4096, 26, 100000, 64
NC, NS, L = 2, 16, 16
NW = NC * NS
RPW = B // NW
CH = 8
RPC = RPW // CH
IPC = RPC * F
SUB = 4
IPS = IPC // SUB
VROW = 32
CW = 2 * K


def _sc_body(idx_hbm, vals_hbm, lin_hbm, cat_hbm, out_hbm,
             idx_v, vals_v, cbuf0, cbuf1, lbuf0, lbuf1,
             outb, sem0, sem1):
    cbufs, lbufs, sems = (cbuf0, cbuf1), (lbuf0, lbuf1), (sem0, sem1)
    wid = lax.axis_index("s") * NC + lax.axis_index("c")

    pltpu.sync_copy(idx_hbm.at[wid], idx_v)
    pltpu.sync_copy(vals_hbm.at[wid], vals_v)

    zf = jnp.zeros((L,), jnp.float32)
    iota = lax.iota(jnp.int32, L)
    riota26 = iota * F
    riota32 = iota * VROW

    def issue(c, b):
        hs = []
        for j in range(SUB):
            isl = idx_v.at[pl.ds((SUB * c + j) * IPS, IPS)]
            dsl = pl.ds(j * IPS, IPS)
            hs.append(pltpu.async_copy(
                cat_hbm.at[isl], cbufs[b].at[dsl], sems[b]))
            hs.append(pltpu.async_copy(
                lin_hbm.at[isl], lbufs[b].at[dsl], sems[b]))
        return hs

    def compute(c, b):
        cb = cbufs[b]
        lb = lbufs[b]

        ssum_vec = zf
        lin_vec = zf
        vgbase = riota32 + (c * RPC * VROW)
        for f in range(F):
            v = plsc.load_gather(vals_v, [vgbase + f])
            lg = plsc.load_gather(lb, [riota26 + f])
            ssum_vec = ssum_vec + v
            lin_vec = lin_vec + v * lg

        zi = jnp.zeros((L,), jnp.int32)
        rot = jnp.bitwise_and(iota + 1, L - 1)

        def row_body(rr, pred_vec):
            vbase = (c * RPC + rr) * VROW
            vrow0 = vals_v[pl.ds(vbase, L)]
            vrow1 = vals_v[pl.ds(vbase + L, L)]
            base = rr * F

            me = [None] * 4
            mu = [None] * 4
            pev = None
            puv = None

            def acc(a, x):
                return x if a is None else a + x

            vrow = vrow0
            for f in range(F):
                if f == L:
                    vrow = vrow1
                v = vrow.at[zi].get(mode="promise_in_bounds")
                vrow = vrow.at[rot].get(mode="promise_in_bounds")
                j = base + f
                for kc in range(4):
                    e = cb[j, pl.ds(kc * L, L)]
                    t = cb[j, pl.ds(K + kc * L, L)]
                    u = e + t
                    es = v * e
                    us = v * u
                    me[kc] = acc(me[kc], es)
                    mu[kc] = acc(mu[kc], us)
                    pev = acc(pev, es * e)
                    puv = acc(puv, us * u)

            ridx = jnp.full((L,), rr, jnp.int32)
            srow = ssum_vec.at[ridx].get(mode="promise_in_bounds")
            linr = lin_vec.at[ridx].get(mode="promise_in_bounds")
            vec = ((pev + puv) * srow
                   - me[0] * me[0] - me[1] * me[1]
                   - me[2] * me[2] - me[3] * me[3]
                   - mu[0] * mu[0] - mu[1] * mu[1]
                   - mu[2] * mu[2] - mu[3] * mu[3])
            red = lax.reduce_sum(vec, axes=(0,))
            return jnp.where(iota == rr, linr + 0.5 * red, pred_vec)

        preds = lax.fori_loop(0, RPC, row_body, zf)
        outb[pl.ds(c * RPC, RPC)] = preds

    pending = issue(0, 0)
    for c in range(CH):
        nxt = issue(c + 1, (c + 1) % 2) if c + 1 < CH else []
        for h in pending:
            h.wait()
        compute(c, c % 2)
        pending = nxt

    pltpu.sync_copy(outb, out_hbm.at[pl.ds(wid * RPW, RPW)])


VB = 4096


def _transcat_body(et_ref, tt_ref, out_ref):
    out_ref[:, :K] = jnp.swapaxes(et_ref[...], 0, 1)
    out_ref[:, K:] = jnp.swapaxes(tt_ref[...], 0, 1)


def _transcat(emb_t, trans_t):
    grid = (V + VB - 1) // VB
    return pl.pallas_call(
        _transcat_body,
        grid=(grid,),
        in_specs=[
            pl.BlockSpec((K, VB), lambda i: (0, i)),
            pl.BlockSpec((K, VB), lambda i: (0, i)),
        ],
        out_specs=pl.BlockSpec((VB, CW), lambda i: (i, 0)),
        out_shape=jax.ShapeDtypeStruct((V, CW), jnp.float32),
    )(emb_t, trans_t)


def kernel(sparse_indices, sparse_values, var_linear, var_emb_factors,
           var_trans_factors):
    idx = sparse_indices.astype(jnp.int32).reshape(NW, RPW * F)
    vals = jnp.pad(sparse_values.astype(jnp.float32),
                   ((0, 0), (0, VROW - F))).reshape(NW, RPW * VROW)
    lin = var_linear.reshape(V)
    cat = _transcat(var_emb_factors.T, var_trans_factors.T)

    run = pl.kernel(
        _sc_body,
        out_type=jax.ShapeDtypeStruct((B,), jnp.float32),
        mesh=plsc.VectorSubcoreMesh(core_axis_name="c", subcore_axis_name="s"),
        compiler_params=pltpu.CompilerParams(
            needs_layout_passes=False, use_tc_tiling_on_sc=False),
        scratch_types=[
            pltpu.VMEM((RPW * F,), jnp.int32),
            pltpu.VMEM((RPW * VROW,), jnp.float32),
            pltpu.VMEM((IPC, CW), jnp.float32),
            pltpu.VMEM((IPC, CW), jnp.float32),
            pltpu.VMEM((IPC,), jnp.float32),
            pltpu.VMEM((IPC,), jnp.float32),
            pltpu.VMEM((RPW,), jnp.float32),
            pltpu.SemaphoreType.DMA,
            pltpu.SemaphoreType.DMA,
        ],
    )
    preds = run(idx, vals, lin, cat)
    return preds.reshape(B, 1)
